# baseline (speedup 1.0000x reference)
import functools

import jax
import jax.numpy as jnp
from jax import lax
from jax.experimental import pallas as pl
from jax.experimental.pallas import tpu as pltpu
from jax.experimental.pallas import tpu_sc as plsc

N = 10000
E = 320000
D = 128
C = 64
NTILES = 16
NP = 10240
RPT = NP // NTILES
CH = 128
NCHT = 158
EPT = NCHT * CH
EP = NTILES * EPT
EPW32 = EP // 32
BN = 1024
NB = NP // BN


def _sc_mesh():
    return plsc.VectorSubcoreMesh(core_axis_name="c", subcore_axis_name="s")


@functools.partial(
    pl.kernel,
    out_type=jax.ShapeDtypeStruct((32, NP), jnp.float32),
    mesh=_sc_mesh(),
    compiler_params=pltpu.CompilerParams(
        needs_layout_passes=False, use_tc_tiling_on_sc=False),
    scratch_types=[
        pltpu.VMEM((EPW32,), jnp.int32),
        pltpu.VMEM((NP,), jnp.float32),
    ],
)
def _deg_kernel(dst_hbm, out_hbm, dstv, degv):
    wid = lax.axis_index("c") * NTILES + lax.axis_index("s")
    pltpu.sync_copy(dst_hbm.at[wid], dstv)

    def zero_body(i, carry):
        degv[pl.ds(i * 16, 16)] = jnp.zeros((16,), jnp.float32)
        return carry

    lax.fori_loop(0, NP // 16, zero_body, 0)

    ones = jnp.ones((16,), jnp.float32)

    def scat_body(i, carry):
        idx = dstv[pl.ds(i * 16, 16)]
        plsc.addupdate_scatter(degv, [idx], ones)
        return carry

    lax.fori_loop(0, EPW32 // 16, scat_body, 0)
    pltpu.sync_copy(degv, out_hbm.at[wid])


def _make_phase(d):
    W = d // 2
    WG = d // 32
    NBUF = 3
    PF = 2

    @functools.partial(
        pl.kernel,
        out_type=[
            jax.ShapeDtypeStruct((2 * NP, d), jnp.float32),
            jax.ShapeDtypeStruct((2 * NP, W), jnp.int32),
        ],
        mesh=_sc_mesh(),
        compiler_params=pltpu.CompilerParams(
            needs_layout_passes=False, use_tc_tiling_on_sc=False),
        scratch_types=[
            pltpu.VMEM((NCHT, CH), jnp.int32),
            pltpu.VMEM((NCHT, CH), jnp.int32),
            pltpu.VMEM((CH, W), jnp.int32),
            pltpu.VMEM((CH, W), jnp.int32),
            pltpu.VMEM((CH, W), jnp.int32),
            pltpu.VMEM((CH, d), jnp.float32),
            pltpu.VMEM((CH, d), jnp.float32),
            pltpu.VMEM((CH, d), jnp.float32),
            pltpu.VMEM((CH, d), jnp.float32),
            pltpu.VMEM((RPT,), jnp.float32),
            pltpu.VMEM_SHARED((NP, d), jnp.float32),
            pltpu.SemaphoreType.DMA,
            pltpu.SemaphoreType.DMA,
            pltpu.SemaphoreType.DMA,
            pltpu.SemaphoreType.DMA,
            pltpu.SemaphoreType.DMA,
            pltpu.SemaphoreType.DMA,
            pltpu.SemaphoreType.DMA,
        ],
    )
    def phase(u_hbm, src_hbm, dst_hbm, dinv2_hbm, out_hbm, u2_hbm,
              srcv, dstv, i0, i1, i2, f0, f1, f2, f3, d2v, acc_sh,
              g0, g1, g2, s0, s1, s2, s3):
        ibufs = (i0, i1, i2)
        fbufs = (f0, f1, f2, f3)
        gsems = (g0, g1, g2)
        ssems = (s0, s1, s2, s3)
        NIB = 3
        NFB = 4
        c = lax.axis_index("c")
        s = lax.axis_index("s")
        row0 = s * RPT
        cnp = c * NP

        pltpu.sync_copy(src_hbm.at[s], srcv)
        pltpu.sync_copy(dst_hbm.at[s], dstv)
        pltpu.sync_copy(dinv2_hbm.at[0, pl.ds(row0, RPT)], d2v)
        coff = (cnp * jnp.ones((16,), jnp.int32)).astype(jnp.int32)

        def off_body(k, carry):
            for j in range(CH // 16):
                sl = pl.ds(j * 16, 16)
                srcv[k, sl] = srcv[k, sl] + coff
            return carry

        lax.fori_loop(0, NCHT, off_body, 0)

        mask_hi = jnp.full((16,), -65536, jnp.int32)

        def unpack_chunk(ib, fb):
            def rows8(r8, carry):
                for dr in range(8):
                    r = r8 * 8 + dr
                    for wb in range(WG):
                        w = ib[r, pl.ds(wb * 16, 16)]
                        fb[r, pl.ds(wb * 32, 16)] = plsc.bitcast(
                            jnp.left_shift(w, 16), jnp.float32)
                        fb[r, pl.ds(wb * 32 + 16, 16)] = plsc.bitcast(
                            jnp.bitwise_and(w, mask_hi), jnp.float32)
                return carry

            lax.fori_loop(0, CH // 8, rows8, 0)

        def init_body(q, carry):
            r0 = q * CH
            pltpu.sync_copy(u_hbm.at[pl.ds(cnp + row0 + r0, CH)], i0)
            unpack_chunk(i0, f0)
            pltpu.sync_copy(f0, acc_sh.at[pl.ds(row0 + r0, CH)])
            return carry

        lax.fori_loop(0, RPT // CH, init_body, 0)
        plsc.subcore_barrier()

        def run_hop(src_ref):
            for k in range(PF):
                pltpu.async_copy(src_ref.at[srcv.at[k]], ibufs[k], gsems[k])

            def pipe_body(kk, carry):
                for off in range(12):
                    k = kk * 12 + off
                    bi = off % NIB
                    bf = off % NFB

                    @pl.when(k < NCHT)
                    def _():
                        pltpu.make_async_copy(
                            src_ref.at[srcv.at[k]], ibufs[bi], gsems[bi]).wait()

                        @pl.when(k >= NFB)
                        def _():
                            pltpu.make_async_copy(
                                fbufs[bf], acc_sh.at[dstv.at[k - NFB]],
                                ssems[bf]).wait()

                        unpack_chunk(ibufs[bi], fbufs[bf])
                        pltpu.async_copy(
                            fbufs[bf], acc_sh.at[dstv.at[k]], ssems[bf],
                            add=True)

                        @pl.when(k + PF < NCHT)
                        def _():
                            bb = (off + PF) % NIB
                            pltpu.async_copy(
                                src_ref.at[srcv.at[k + PF]], ibufs[bb],
                                gsems[bb])
                return carry

            lax.fori_loop(0, (NCHT + 11) // 12, pipe_body, 0)
            for j in range(NCHT - NFB, NCHT):
                pltpu.make_async_copy(
                    fbufs[j % NFB], acc_sh.at[dstv.at[j]],
                    ssems[j % NFB]).wait()

        run_hop(u_hbm)
        plsc.subcore_barrier()

        def scale_chunk(q, carry):
            r0 = q * CH
            pltpu.sync_copy(acc_sh.at[pl.ds(row0 + r0, CH)], f0)

            def grp(g, carry2):
                base = g * 16
                vec = d2v[pl.ds(r0 + base, 16)]
                for i in range(16):
                    val = vec[i]
                    row = base + i
                    for j in range(d // 16):
                        sl = pl.ds(j * 16, 16)
                        f0[row, sl] = f0[row, sl] * val
                    for wb in range(WG):
                        a = plsc.bitcast(
                            f0[row, pl.ds(wb * 32, 16)], jnp.int32) + 32768
                        bi = plsc.bitcast(
                            f0[row, pl.ds(wb * 32 + 16, 16)], jnp.int32) + 32768
                        w = jnp.bitwise_or(
                            jax.lax.shift_right_logical(a, 16),
                            jnp.left_shift(
                                jax.lax.shift_right_logical(bi, 16), 16))
                        i0[row, pl.ds(wb * 16, 16)] = w
                return carry2

            lax.fori_loop(0, CH // 16, grp, 0)
            pltpu.sync_copy(f0, acc_sh.at[pl.ds(row0 + r0, CH)])
            pltpu.sync_copy(i0, u2_hbm.at[pl.ds(cnp + row0 + r0, CH)])
            return carry

        lax.fori_loop(0, RPT // CH, scale_chunk, 0)
        plsc.subcore_barrier()

        run_hop(u2_hbm)
        plsc.subcore_barrier()

        def wb_body(q, carry):
            r0 = q * CH
            pltpu.sync_copy(acc_sh.at[pl.ds(row0 + r0, CH)], f0)
            pltpu.sync_copy(f0, out_hbm.at[pl.ds(cnp + row0 + r0, CH)])
            return carry

        lax.fori_loop(0, RPT // CH, wb_body, 0)

    return phase


_phase_a = _make_phase(C)
_phase_b = _make_phase(C // 2)


def _dinv_of(degp_blk):
    deg = jnp.sum(degp_blk, axis=0) + 1.0
    return lax.rsqrt(deg), deg


def _pack_rows(u):
    ui = lax.bitcast_convert_type(u, jnp.int32) + 32768
    parts = []
    for g in range(u.shape[-1] // 32):
        a = ui[..., g * 32: g * 32 + 16]
        b = ui[..., g * 32 + 16: g * 32 + 32]
        parts.append(jnp.bitwise_or(
            lax.shift_right_logical(a, 16),
            jnp.left_shift(lax.shift_right_logical(b, 16), 16)))
    return jnp.concatenate(parts, axis=-1)


def _prep_body(x_ref, w_ref, degp_ref, u_ref, d2_ref):
    dinv, deg = _dinv_of(degp_ref[...])
    y = jnp.dot(x_ref[0], w_ref[0], preferred_element_type=jnp.float32)
    u_ref[0] = _pack_rows(y * dinv[:, None])
    d2_ref[0] = 1.0 / deg


def _tc_prep(xp, wstk, degp):
    return pl.pallas_call(
        _prep_body,
        grid=(2, NB),
        in_specs=[
            pl.BlockSpec((1, BN, D), lambda i, j: (i, j, 0)),
            pl.BlockSpec((1, D, C), lambda i, j: (i, 0, 0)),
            pl.BlockSpec((32, BN), lambda i, j: (0, j)),
        ],
        out_specs=[
            pl.BlockSpec((1, BN, C // 2), lambda i, j: (i, j, 0)),
            pl.BlockSpec((1, BN), lambda i, j: (0, j)),
        ],
        out_shape=[
            jax.ShapeDtypeStruct((2, NP, C // 2), jnp.int32),
            jax.ShapeDtypeStruct((1, NP), jnp.float32),
        ],
    )(xp, wstk, degp)


def _mid_body(acc_ref, degp_ref, b_ref, wf_ref, uz_ref):
    dinv, _ = _dinv_of(degp_ref[...])
    h0 = jnp.maximum(acc_ref[0] * dinv[:, None] + b_ref[0], 0.0)
    h1 = jnp.maximum(acc_ref[1] * dinv[:, None] + b_ref[1], 0.0)
    z = (jnp.dot(h0, wf_ref[:C], preferred_element_type=jnp.float32)
         + jnp.dot(h1, wf_ref[C:], preferred_element_type=jnp.float32))
    uz = z * dinv[:, None]
    uz_ref[0] = _pack_rows(uz[:, : C // 2])
    uz_ref[1] = _pack_rows(uz[:, C // 2:])


def _tc_mid(accA, degp, bstk, wf):
    return pl.pallas_call(
        _mid_body,
        grid=(NB,),
        in_specs=[
            pl.BlockSpec((2, BN, C), lambda j: (0, j, 0)),
            pl.BlockSpec((32, BN), lambda j: (0, j)),
            pl.BlockSpec((2, 1, C), lambda j: (0, 0, 0)),
            pl.BlockSpec((D, C), lambda j: (0, 0)),
        ],
        out_specs=pl.BlockSpec((2, BN, C // 4), lambda j: (0, j, 0)),
        out_shape=jax.ShapeDtypeStruct((2, NP, C // 4), jnp.int32),
    )(accA, degp, bstk, wf)


def _final_body(acc_ref, degp_ref, bf_ref, out_ref):
    dinv, _ = _dinv_of(degp_ref[...])
    y = jnp.concatenate([acc_ref[0], acc_ref[1]], axis=-1)
    out_ref[...] = y * dinv[:, None] + bf_ref[0]


def _tc_final(accB, degp, bf2d):
    return pl.pallas_call(
        _final_body,
        grid=(NB,),
        in_specs=[
            pl.BlockSpec((2, BN, C // 2), lambda j: (0, j, 0)),
            pl.BlockSpec((32, BN), lambda j: (0, j)),
            pl.BlockSpec((1, C), lambda j: (0, 0)),
        ],
        out_specs=pl.BlockSpec((BN, C), lambda j: (j, 0)),
        out_shape=jax.ShapeDtypeStruct((NP, C), jnp.float32),
    )(accB, degp, bf2d)


def kernel(x_list, edge_index, W0, b0, W1, b1, Wf, bf):
    src = edge_index[0]
    dst = edge_index[1]
    pad = jnp.full((EP - E,), NP - 1, dtype=jnp.int32)
    srcp = jnp.concatenate([src, pad])
    dstp = jnp.concatenate([dst, pad])
    srcA = srcp.reshape(NTILES, NCHT, CH)
    dstA = dstp.reshape(NTILES, NCHT, CH)
    dst32 = dstp.reshape(32, EPW32)

    xp = jnp.pad(x_list, ((0, 0), (0, NP - N), (0, 0)))
    wstk = jnp.stack([W0, W1])
    bstk = jnp.stack([b0, b1])[:, None, :]

    degp = _deg_kernel(dst32)
    u, dinv2 = _tc_prep(xp, wstk, degp)
    accA, _ = _phase_a(u.reshape(2 * NP, C // 2), srcA, dstA, dinv2)
    uz = _tc_mid(accA.reshape(2, NP, C), degp, bstk, Wf)
    accB, _ = _phase_b(uz.reshape(2 * NP, C // 4), srcA, dstA, dinv2)
    out = _tc_final(accB.reshape(2, NP, C // 2), degp, bf[None, :])
    return out[:N]


# speedup vs baseline: 1.1095x; 1.1095x over previous
import functools

import jax
import jax.numpy as jnp
from jax import lax
from jax.experimental import pallas as pl
from jax.experimental.pallas import tpu as pltpu
from jax.experimental.pallas import tpu_sc as plsc

N = 10000
E = 320000
D = 128
C = 64
NTILES = 16
NP = 10240
RPT = NP // NTILES
CH = 128
NCHT = 158
EPT = NCHT * CH
EP = NTILES * EPT
EPW32 = EP // 32
BN = 1024
NB = NP // BN


def _sc_mesh():
    return plsc.VectorSubcoreMesh(core_axis_name="c", subcore_axis_name="s")


@functools.partial(
    pl.kernel,
    out_type=jax.ShapeDtypeStruct((32, NP), jnp.float32),
    mesh=_sc_mesh(),
    compiler_params=pltpu.CompilerParams(
        needs_layout_passes=False, use_tc_tiling_on_sc=False),
    scratch_types=[
        pltpu.VMEM((EPW32,), jnp.int32),
        pltpu.VMEM((NP,), jnp.float32),
    ],
)
def _deg_kernel(dst_hbm, out_hbm, dstv, degv):
    wid = lax.axis_index("c") * NTILES + lax.axis_index("s")
    pltpu.sync_copy(dst_hbm.at[wid], dstv)

    def zero_body(i, carry):
        degv[pl.ds(i * 16, 16)] = jnp.zeros((16,), jnp.float32)
        return carry

    lax.fori_loop(0, NP // 16, zero_body, 0)

    ones = jnp.ones((16,), jnp.float32)

    def scat_body(i, carry):
        idx = dstv[pl.ds(i * 16, 16)]
        plsc.addupdate_scatter(degv, [idx], ones)
        return carry

    lax.fori_loop(0, EPW32 // 16, scat_body, 0)
    pltpu.sync_copy(degv, out_hbm.at[wid])


def _make_phase(d):
    W = d // 2
    WG = d // 32
    NBUF = 3
    PF = 2

    @functools.partial(
        pl.kernel,
        out_type=[
            jax.ShapeDtypeStruct((2 * NP, d), jnp.float32),
            jax.ShapeDtypeStruct((2 * NP, W), jnp.int32),
        ],
        mesh=_sc_mesh(),
        compiler_params=pltpu.CompilerParams(
            needs_layout_passes=False, use_tc_tiling_on_sc=False),
        scratch_types=[
            pltpu.VMEM((NCHT, CH), jnp.int32),
            pltpu.VMEM((NCHT, CH), jnp.int32),
            pltpu.VMEM((CH, W), jnp.int32),
            pltpu.VMEM((CH, W), jnp.int32),
            pltpu.VMEM((CH, d), jnp.float32),
            pltpu.VMEM((CH, d), jnp.float32),
            pltpu.VMEM((RPT,), jnp.float32),
            pltpu.VMEM_SHARED((NP, d), jnp.float32),
            pltpu.SemaphoreType.DMA,
            pltpu.SemaphoreType.DMA,
            pltpu.SemaphoreType.DMA,
            pltpu.SemaphoreType.DMA,
        ],
    )
    def phase(u_hbm, src_hbm, dst_hbm, dinv2_hbm, out_hbm, u2_hbm,
              srcv, dstv, i0, i1, f0, f1, d2v, acc_sh,
              g0, g1, s0, s1):
        ibufs = (i0, i1)
        fbufs = (f0, f1)
        gsems = (g0, g1)
        ssems = (s0, s1)
        c = lax.axis_index("c")
        s = lax.axis_index("s")
        row0 = s * RPT
        cnp = c * NP

        pltpu.sync_copy(src_hbm.at[s], srcv)
        pltpu.sync_copy(dst_hbm.at[s], dstv)
        pltpu.sync_copy(dinv2_hbm.at[0, pl.ds(row0, RPT)], d2v)
        coff = (cnp * jnp.ones((16,), jnp.int32)).astype(jnp.int32)

        def off_body(k, carry):
            for j in range(CH // 16):
                sl = pl.ds(j * 16, 16)
                srcv[k, sl] = srcv[k, sl] + coff
            return carry

        lax.fori_loop(0, NCHT, off_body, 0)

        mask_hi = jnp.full((16,), -65536, jnp.int32)

        def unpack_chunk(ib, fb):
            for r in range(CH):
                for wb in range(WG):
                    w = ib[r, pl.ds(wb * 16, 16)]
                    fb[r, pl.ds(wb * 32, 16)] = plsc.bitcast(
                        jnp.left_shift(w, 16), jnp.float32)
                    fb[r, pl.ds(wb * 32 + 16, 16)] = plsc.bitcast(
                        jnp.bitwise_and(w, mask_hi), jnp.float32)

        def init_body(q, carry):
            r0 = q * CH
            pltpu.sync_copy(u_hbm.at[pl.ds(cnp + row0 + r0, CH)], i0)
            unpack_chunk(i0, f0)
            pltpu.sync_copy(f0, acc_sh.at[pl.ds(row0 + r0, CH)])
            return carry

        lax.fori_loop(0, RPT // CH, init_body, 0)
        plsc.subcore_barrier()

        def run_hop(src_ref):
            pltpu.async_copy(src_ref.at[srcv.at[0]], ibufs[0], gsems[0])

            def pipe_body(kk, carry):
                for off in range(2):
                    k = kk * 2 + off
                    bi = off
                    bf = off

                    @pl.when(k < NCHT)
                    def _():
                        pltpu.make_async_copy(
                            src_ref.at[srcv.at[k]], ibufs[bi], gsems[bi]).wait()

                        @pl.when(k + 1 < NCHT)
                        def _():
                            pltpu.async_copy(
                                src_ref.at[srcv.at[k + 1]], ibufs[1 - bi],
                                gsems[1 - bi])

                        @pl.when(k >= 2)
                        def _():
                            pltpu.make_async_copy(
                                fbufs[bf], acc_sh.at[dstv.at[k - 2]],
                                ssems[bf]).wait()

                        unpack_chunk(ibufs[bi], fbufs[bf])
                        pltpu.async_copy(
                            fbufs[bf], acc_sh.at[dstv.at[k]], ssems[bf],
                            add=True)
                return carry

            lax.fori_loop(0, (NCHT + 1) // 2, pipe_body, 0)
            for j in range(NCHT - 2, NCHT):
                pltpu.make_async_copy(
                    fbufs[j % 2], acc_sh.at[dstv.at[j]], ssems[j % 2]).wait()

        run_hop(u_hbm)
        plsc.subcore_barrier()

        def scale_chunk(q, carry):
            r0 = q * CH
            pltpu.sync_copy(acc_sh.at[pl.ds(row0 + r0, CH)], f0)

            def grp(g, carry2):
                base = g * 16
                vec = d2v[pl.ds(r0 + base, 16)]
                for i in range(16):
                    val = vec[i]
                    row = base + i
                    for j in range(d // 16):
                        sl = pl.ds(j * 16, 16)
                        f0[row, sl] = f0[row, sl] * val
                    for wb in range(WG):
                        a = plsc.bitcast(
                            f0[row, pl.ds(wb * 32, 16)], jnp.int32) + 32768
                        bi = plsc.bitcast(
                            f0[row, pl.ds(wb * 32 + 16, 16)], jnp.int32) + 32768
                        w = jnp.bitwise_or(
                            jax.lax.shift_right_logical(a, 16),
                            jnp.left_shift(
                                jax.lax.shift_right_logical(bi, 16), 16))
                        i0[row, pl.ds(wb * 16, 16)] = w
                return carry2

            lax.fori_loop(0, CH // 16, grp, 0)
            pltpu.sync_copy(f0, acc_sh.at[pl.ds(row0 + r0, CH)])
            pltpu.sync_copy(i0, u2_hbm.at[pl.ds(cnp + row0 + r0, CH)])
            return carry

        lax.fori_loop(0, RPT // CH, scale_chunk, 0)
        plsc.subcore_barrier()

        run_hop(u2_hbm)
        plsc.subcore_barrier()

        def wb_body(q, carry):
            r0 = q * CH
            pltpu.sync_copy(acc_sh.at[pl.ds(row0 + r0, CH)], f0)
            pltpu.sync_copy(f0, out_hbm.at[pl.ds(cnp + row0 + r0, CH)])
            return carry

        lax.fori_loop(0, RPT // CH, wb_body, 0)

    return phase


_phase_a = _make_phase(C)
_phase_b = _make_phase(C // 2)


def _dinv_of(degp_blk):
    deg = jnp.sum(degp_blk, axis=0) + 1.0
    return lax.rsqrt(deg), deg


def _pack_rows(u):
    ui = lax.bitcast_convert_type(u, jnp.int32) + 32768
    parts = []
    for g in range(u.shape[-1] // 32):
        a = ui[..., g * 32: g * 32 + 16]
        b = ui[..., g * 32 + 16: g * 32 + 32]
        parts.append(jnp.bitwise_or(
            lax.shift_right_logical(a, 16),
            jnp.left_shift(lax.shift_right_logical(b, 16), 16)))
    return jnp.concatenate(parts, axis=-1)


def _prep_body(x_ref, w_ref, degp_ref, u_ref, d2_ref):
    dinv, deg = _dinv_of(degp_ref[...])
    y = jnp.dot(x_ref[0], w_ref[0], preferred_element_type=jnp.float32)
    u_ref[0] = _pack_rows(y * dinv[:, None])
    d2_ref[0] = 1.0 / deg


def _tc_prep(xp, wstk, degp):
    return pl.pallas_call(
        _prep_body,
        grid=(2, NB),
        in_specs=[
            pl.BlockSpec((1, BN, D), lambda i, j: (i, j, 0)),
            pl.BlockSpec((1, D, C), lambda i, j: (i, 0, 0)),
            pl.BlockSpec((32, BN), lambda i, j: (0, j)),
        ],
        out_specs=[
            pl.BlockSpec((1, BN, C // 2), lambda i, j: (i, j, 0)),
            pl.BlockSpec((1, BN), lambda i, j: (0, j)),
        ],
        out_shape=[
            jax.ShapeDtypeStruct((2, NP, C // 2), jnp.int32),
            jax.ShapeDtypeStruct((1, NP), jnp.float32),
        ],
    )(xp, wstk, degp)


def _mid_body(acc_ref, degp_ref, b_ref, wf_ref, uz_ref):
    dinv, _ = _dinv_of(degp_ref[...])
    h0 = jnp.maximum(acc_ref[0] * dinv[:, None] + b_ref[0], 0.0)
    h1 = jnp.maximum(acc_ref[1] * dinv[:, None] + b_ref[1], 0.0)
    z = (jnp.dot(h0, wf_ref[:C], preferred_element_type=jnp.float32)
         + jnp.dot(h1, wf_ref[C:], preferred_element_type=jnp.float32))
    uz = z * dinv[:, None]
    uz_ref[0] = _pack_rows(uz[:, : C // 2])
    uz_ref[1] = _pack_rows(uz[:, C // 2:])


def _tc_mid(accA, degp, bstk, wf):
    return pl.pallas_call(
        _mid_body,
        grid=(NB,),
        in_specs=[
            pl.BlockSpec((2, BN, C), lambda j: (0, j, 0)),
            pl.BlockSpec((32, BN), lambda j: (0, j)),
            pl.BlockSpec((2, 1, C), lambda j: (0, 0, 0)),
            pl.BlockSpec((D, C), lambda j: (0, 0)),
        ],
        out_specs=pl.BlockSpec((2, BN, C // 4), lambda j: (0, j, 0)),
        out_shape=jax.ShapeDtypeStruct((2, NP, C // 4), jnp.int32),
    )(accA, degp, bstk, wf)


def _final_body(acc_ref, degp_ref, bf_ref, out_ref):
    dinv, _ = _dinv_of(degp_ref[...])
    y = jnp.concatenate([acc_ref[0], acc_ref[1]], axis=-1)
    out_ref[...] = y * dinv[:, None] + bf_ref[0]


def _tc_final(accB, degp, bf2d):
    return pl.pallas_call(
        _final_body,
        grid=(NB,),
        in_specs=[
            pl.BlockSpec((2, BN, C // 2), lambda j: (0, j, 0)),
            pl.BlockSpec((32, BN), lambda j: (0, j)),
            pl.BlockSpec((1, C), lambda j: (0, 0)),
        ],
        out_specs=pl.BlockSpec((BN, C), lambda j: (j, 0)),
        out_shape=jax.ShapeDtypeStruct((NP, C), jnp.float32),
    )(accB, degp, bf2d)


def kernel(x_list, edge_index, W0, b0, W1, b1, Wf, bf):
    src = edge_index[0]
    dst = edge_index[1]
    pad = jnp.full((EP - E,), NP - 1, dtype=jnp.int32)
    srcp = jnp.concatenate([src, pad])
    dstp = jnp.concatenate([dst, pad])
    srcA = srcp.reshape(NTILES, NCHT, CH)
    dstA = dstp.reshape(NTILES, NCHT, CH)
    dst32 = dstp.reshape(32, EPW32)

    xp = jnp.pad(x_list, ((0, 0), (0, NP - N), (0, 0)))
    wstk = jnp.stack([W0, W1])
    bstk = jnp.stack([b0, b1])[:, None, :]

    degp = _deg_kernel(dst32)
    u, dinv2 = _tc_prep(xp, wstk, degp)
    accA, _ = _phase_a(u.reshape(2 * NP, C // 2), srcA, dstA, dinv2)
    uz = _tc_mid(accA.reshape(2, NP, C), degp, bstk, Wf)
    accB, _ = _phase_b(uz.reshape(2 * NP, C // 4), srcA, dstA, dinv2)
    out = _tc_final(accB.reshape(2, NP, C // 2), degp, bf[None, :])
    return out[:N]


# speedup vs baseline: 1.1716x; 1.0560x over previous
import functools

import jax
import jax.numpy as jnp
from jax import lax
from jax.experimental import pallas as pl
from jax.experimental.pallas import tpu as pltpu
from jax.experimental.pallas import tpu_sc as plsc

N = 10000
E = 320000
D = 128
C = 64
NTILES = 16
NP = 10240
RPT = NP // NTILES
CH = 128
NCHT = 158
EPT = NCHT * CH
EP = NTILES * EPT
EPW32 = EP // 32
BN = 1024
NB = NP // BN


def _sc_mesh():
    return plsc.VectorSubcoreMesh(core_axis_name="c", subcore_axis_name="s")


@functools.partial(
    pl.kernel,
    out_type=jax.ShapeDtypeStruct((32, NP), jnp.float32),
    mesh=_sc_mesh(),
    compiler_params=pltpu.CompilerParams(
        needs_layout_passes=False, use_tc_tiling_on_sc=False),
    scratch_types=[
        pltpu.VMEM((EPW32,), jnp.int32),
        pltpu.VMEM((NP,), jnp.float32),
    ],
)
def _deg_kernel(dst_hbm, out_hbm, dstv, degv):
    wid = lax.axis_index("c") * NTILES + lax.axis_index("s")
    pltpu.sync_copy(dst_hbm.at[wid], dstv)

    def zero_body(i, carry):
        degv[pl.ds(i * 16, 16)] = jnp.zeros((16,), jnp.float32)
        return carry

    lax.fori_loop(0, NP // 16, zero_body, 0)

    ones = jnp.ones((16,), jnp.float32)

    def scat_body(i, carry):
        idx = dstv[pl.ds(i * 16, 16)]
        plsc.addupdate_scatter(degv, [idx], ones)
        return carry

    lax.fori_loop(0, EPW32 // 16, scat_body, 0)
    pltpu.sync_copy(degv, out_hbm.at[wid])


def _make_phase(d):
    W = d // 2
    WG = d // 32
    NBUF = 3
    PF = 2

    @functools.partial(
        pl.kernel,
        out_type=[
            jax.ShapeDtypeStruct((2 * NP, d), jnp.float32),
            jax.ShapeDtypeStruct((2 * NP, W), jnp.int32),
        ],
        mesh=_sc_mesh(),
        compiler_params=pltpu.CompilerParams(
            needs_layout_passes=False, use_tc_tiling_on_sc=False),
        scratch_types=[
            pltpu.VMEM((NCHT, CH), jnp.int32),
            pltpu.VMEM((NCHT, CH), jnp.int32),
            pltpu.VMEM((CH, W), jnp.int32),
            pltpu.VMEM((CH, W), jnp.int32),
            pltpu.VMEM((CH, d), jnp.float32),
            pltpu.VMEM((CH, d), jnp.float32),
            pltpu.VMEM((RPT,), jnp.float32),
            pltpu.VMEM_SHARED((NP, d), jnp.float32),
            pltpu.SemaphoreType.DMA,
            pltpu.SemaphoreType.DMA,
            pltpu.SemaphoreType.DMA,
            pltpu.SemaphoreType.DMA,
        ],
    )
    def phase(u_hbm, src_hbm, dst_hbm, dinv2_hbm, out_hbm, u2_hbm,
              srcv, dstv, i0, i1, f0, f1, d2v, acc_sh,
              g0, g1, s0, s1):
        ibufs = (i0, i1)
        fbufs = (f0, f1)
        gsems = (g0, g1)
        ssems = (s0, s1)
        c = lax.axis_index("c")
        s = lax.axis_index("s")
        row0 = s * RPT
        cnp = c * NP

        pltpu.sync_copy(src_hbm.at[s], srcv)
        pltpu.sync_copy(dst_hbm.at[s], dstv)
        pltpu.sync_copy(dinv2_hbm.at[0, pl.ds(row0, RPT)], d2v)
        coff = (cnp * jnp.ones((16,), jnp.int32)).astype(jnp.int32)

        def off_body(k, carry):
            for j in range(CH // 16):
                sl = pl.ds(j * 16, 16)
                srcv[k, sl] = srcv[k, sl] + coff
            return carry

        lax.fori_loop(0, NCHT, off_body, 0)

        mask_hi = jnp.full((16,), -65536, jnp.int32)

        def unpack_chunk(ib, fb):
            for r in range(CH):
                for wb in range(WG):
                    w = ib[r, pl.ds(wb * 16, 16)]
                    fb[r, pl.ds(wb * 32, 16)] = plsc.bitcast(
                        jnp.left_shift(w, 16), jnp.float32)
                    fb[r, pl.ds(wb * 32 + 16, 16)] = plsc.bitcast(
                        jnp.bitwise_and(w, mask_hi), jnp.float32)

        def init_body(q, carry):
            r0 = q * CH
            pltpu.sync_copy(u_hbm.at[pl.ds(cnp + row0 + r0, CH)], i0)
            unpack_chunk(i0, f0)
            pltpu.sync_copy(f0, acc_sh.at[pl.ds(row0 + r0, CH)])
            return carry

        lax.fori_loop(0, RPT // CH, init_body, 0)
        plsc.subcore_barrier()

        def run_hop(src_ref):
            pltpu.async_copy(src_ref.at[srcv.at[0]], ibufs[0], gsems[0])

            def pipe_body(kk, carry):
                for off in range(2):
                    k = kk * 2 + off
                    bi = off
                    bf = off

                    @pl.when(k < NCHT)
                    def _():
                        pltpu.make_async_copy(
                            src_ref.at[srcv.at[k]], ibufs[bi], gsems[bi]).wait()

                        @pl.when(k + 1 < NCHT)
                        def _():
                            pltpu.async_copy(
                                src_ref.at[srcv.at[k + 1]], ibufs[1 - bi],
                                gsems[1 - bi])

                        @pl.when(k >= 2)
                        def _():
                            pltpu.make_async_copy(
                                fbufs[bf], acc_sh.at[dstv.at[k - 2]],
                                ssems[bf]).wait()

                        unpack_chunk(ibufs[bi], fbufs[bf])
                        pltpu.async_copy(
                            fbufs[bf], acc_sh.at[dstv.at[k]], ssems[bf],
                            add=True)
                return carry

            lax.fori_loop(0, (NCHT + 1) // 2, pipe_body, 0)
            for j in range(NCHT - 2, NCHT):
                pltpu.make_async_copy(
                    fbufs[j % 2], acc_sh.at[dstv.at[j]], ssems[j % 2]).wait()

        run_hop(u_hbm)
        plsc.subcore_barrier()

        def scale_chunk(q, carry):
            r0 = q * CH
            pltpu.sync_copy(acc_sh.at[pl.ds(row0 + r0, CH)], f0)

            def grp(g, carry2):
                base = g * 16
                vec = d2v[pl.ds(r0 + base, 16)]
                for i in range(16):
                    val = vec[i]
                    row = base + i
                    for j in range(d // 16):
                        sl = pl.ds(j * 16, 16)
                        f0[row, sl] = f0[row, sl] * val
                    for wb in range(WG):
                        a = plsc.bitcast(
                            f0[row, pl.ds(wb * 32, 16)], jnp.int32) + 32768
                        bi = plsc.bitcast(
                            f0[row, pl.ds(wb * 32 + 16, 16)], jnp.int32) + 32768
                        w = jnp.bitwise_or(
                            jax.lax.shift_right_logical(a, 16),
                            jnp.left_shift(
                                jax.lax.shift_right_logical(bi, 16), 16))
                        i0[row, pl.ds(wb * 16, 16)] = w
                return carry2

            lax.fori_loop(0, CH // 16, grp, 0)
            pltpu.sync_copy(f0, acc_sh.at[pl.ds(row0 + r0, CH)])
            pltpu.sync_copy(i0, u2_hbm.at[pl.ds(cnp + row0 + r0, CH)])
            return carry

        lax.fori_loop(0, RPT // CH, scale_chunk, 0)
        plsc.subcore_barrier()

        run_hop(u2_hbm)
        plsc.subcore_barrier()

        def wb_body(q, carry):
            r0 = q * CH
            pltpu.sync_copy(acc_sh.at[pl.ds(row0 + r0, CH)], f0)
            pltpu.sync_copy(f0, out_hbm.at[pl.ds(cnp + row0 + r0, CH)])
            return carry

        lax.fori_loop(0, RPT // CH, wb_body, 0)

    return phase


def _make_phase_f32(d):
    @functools.partial(
        pl.kernel,
        out_type=jax.ShapeDtypeStruct((2 * NP, d), jnp.float32),
        mesh=_sc_mesh(),
        compiler_params=pltpu.CompilerParams(
            needs_layout_passes=False, use_tc_tiling_on_sc=False),
        scratch_types=[
            pltpu.VMEM((NCHT, CH), jnp.int32),
            pltpu.VMEM((NCHT, CH), jnp.int32),
            pltpu.VMEM((CH, d), jnp.float32),
            pltpu.VMEM((CH, d), jnp.float32),
            pltpu.VMEM((CH, d), jnp.float32),
            pltpu.VMEM((CH, d), jnp.float32),
            pltpu.VMEM((RPT,), jnp.float32),
            pltpu.VMEM_SHARED((NP, d), jnp.float32),
            pltpu.SemaphoreType.DMA,
            pltpu.SemaphoreType.DMA,
            pltpu.SemaphoreType.DMA,
            pltpu.SemaphoreType.DMA,
            pltpu.SemaphoreType.DMA,
            pltpu.SemaphoreType.DMA,
            pltpu.SemaphoreType.DMA,
            pltpu.SemaphoreType.DMA,
        ],
    )
    def phase(u_hbm, src_hbm, dst_hbm, dinv2_hbm, out_hbm,
              srcv, dstv, b0, b1, b2, b3, d2v, acc_sh,
              g0, g1, g2, g3, s0, s1, s2, s3):
        bufs = (b0, b1, b2, b3)
        rowbuf = b0
        gsems = (g0, g1, g2, g3)
        ssems = (s0, s1, s2, s3)
        NBUF = 4
        PF = 2
        c = lax.axis_index("c")
        s = lax.axis_index("s")
        row0 = s * RPT
        cnp = c * NP

        pltpu.sync_copy(src_hbm.at[s], srcv)
        pltpu.sync_copy(dst_hbm.at[s], dstv)
        pltpu.sync_copy(dinv2_hbm.at[0, pl.ds(row0, RPT)], d2v)
        coff = (cnp * jnp.ones((16,), jnp.int32)).astype(jnp.int32)

        def off_body(k, carry):
            for j in range(CH // 16):
                sl = pl.ds(j * 16, 16)
                srcv[k, sl] = srcv[k, sl] + coff
            return carry

        lax.fori_loop(0, NCHT, off_body, 0)

        def init_body(q, carry):
            r0 = q * CH
            pltpu.sync_copy(u_hbm.at[pl.ds(cnp + row0 + r0, CH)], rowbuf)
            pltpu.sync_copy(rowbuf, acc_sh.at[pl.ds(row0 + r0, CH)])
            return carry

        lax.fori_loop(0, RPT // CH, init_body, 0)
        plsc.subcore_barrier()

        def run_hop(src_ref):
            for k in range(PF):
                pltpu.async_copy(src_ref.at[srcv.at[k]], bufs[k], gsems[k])

            def quad_body(kk, carry):
                for off in range(NBUF):
                    k = kk * NBUF + off
                    b = off
                    bb = (off + PF) % NBUF

                    @pl.when(k < NCHT)
                    def _():
                        pltpu.make_async_copy(
                            src_ref.at[srcv.at[k]], bufs[b], gsems[b]).wait()
                        pltpu.async_copy(
                            bufs[b], acc_sh.at[dstv.at[k]], ssems[b], add=True)

                        @pl.when(k + PF < NCHT)
                        def _():
                            @pl.when(k + PF >= NBUF)
                            def _():
                                pltpu.make_async_copy(
                                    bufs[bb], acc_sh.at[dstv.at[k + PF - NBUF]],
                                    ssems[bb]).wait()

                            pltpu.async_copy(
                                src_ref.at[srcv.at[k + PF]], bufs[bb], gsems[bb])
                return carry

            lax.fori_loop(0, (NCHT + NBUF - 1) // NBUF, quad_body, 0)
            for j in range(NCHT - NBUF, NCHT):
                pltpu.make_async_copy(
                    bufs[j % NBUF], acc_sh.at[dstv.at[j]], ssems[j % NBUF]).wait()

        run_hop(u_hbm)
        plsc.subcore_barrier()

        def scale_chunk(q, carry):
            r0 = q * CH
            pltpu.sync_copy(acc_sh.at[pl.ds(row0 + r0, CH)], rowbuf)

            def grp(g, carry2):
                base = g * 16
                vec = d2v[pl.ds(r0 + base, 16)]
                for i in range(16):
                    val = vec[i]
                    for j in range(d // 16):
                        sl = pl.ds(j * 16, 16)
                        rowbuf[base + i, sl] = rowbuf[base + i, sl] * val
                return carry2

            lax.fori_loop(0, CH // 16, grp, 0)
            pltpu.sync_copy(rowbuf, out_hbm.at[pl.ds(cnp + row0 + r0, CH)])
            pltpu.sync_copy(rowbuf, acc_sh.at[pl.ds(row0 + r0, CH)])
            return carry

        lax.fori_loop(0, RPT // CH, scale_chunk, 0)
        plsc.subcore_barrier()

        run_hop(out_hbm)
        plsc.subcore_barrier()

        def wb_body(q, carry):
            r0 = q * CH
            pltpu.sync_copy(acc_sh.at[pl.ds(row0 + r0, CH)], rowbuf)
            pltpu.sync_copy(rowbuf, out_hbm.at[pl.ds(cnp + row0 + r0, CH)])
            return carry

        lax.fori_loop(0, RPT // CH, wb_body, 0)

    return phase


_phase_a = _make_phase(C)
_phase_b = _make_phase_f32(C // 2)


def _dinv_of(degp_blk):
    deg = jnp.sum(degp_blk, axis=0) + 1.0
    return lax.rsqrt(deg), deg


def _pack_rows(u):
    ui = lax.bitcast_convert_type(u, jnp.int32) + 32768
    parts = []
    for g in range(u.shape[-1] // 32):
        a = ui[..., g * 32: g * 32 + 16]
        b = ui[..., g * 32 + 16: g * 32 + 32]
        parts.append(jnp.bitwise_or(
            lax.shift_right_logical(a, 16),
            jnp.left_shift(lax.shift_right_logical(b, 16), 16)))
    return jnp.concatenate(parts, axis=-1)


def _prep_body(x_ref, w_ref, degp_ref, u_ref, d2_ref):
    dinv, deg = _dinv_of(degp_ref[...])
    y = jnp.dot(x_ref[0], w_ref[0], preferred_element_type=jnp.float32)
    u_ref[0] = _pack_rows(y * dinv[:, None])
    d2_ref[0] = 1.0 / deg


def _tc_prep(xp, wstk, degp):
    return pl.pallas_call(
        _prep_body,
        grid=(2, NB),
        in_specs=[
            pl.BlockSpec((1, BN, D), lambda i, j: (i, j, 0)),
            pl.BlockSpec((1, D, C), lambda i, j: (i, 0, 0)),
            pl.BlockSpec((32, BN), lambda i, j: (0, j)),
        ],
        out_specs=[
            pl.BlockSpec((1, BN, C // 2), lambda i, j: (i, j, 0)),
            pl.BlockSpec((1, BN), lambda i, j: (0, j)),
        ],
        out_shape=[
            jax.ShapeDtypeStruct((2, NP, C // 2), jnp.int32),
            jax.ShapeDtypeStruct((1, NP), jnp.float32),
        ],
    )(xp, wstk, degp)


def _mid_body(acc_ref, degp_ref, b_ref, wf_ref, uz_ref):
    dinv, _ = _dinv_of(degp_ref[...])
    h0 = jnp.maximum(acc_ref[0] * dinv[:, None] + b_ref[0], 0.0)
    h1 = jnp.maximum(acc_ref[1] * dinv[:, None] + b_ref[1], 0.0)
    z = (jnp.dot(h0, wf_ref[:C], preferred_element_type=jnp.float32)
         + jnp.dot(h1, wf_ref[C:], preferred_element_type=jnp.float32))
    uz = z * dinv[:, None]
    uz_ref[0] = uz[:, : C // 2]
    uz_ref[1] = uz[:, C // 2:]


def _tc_mid(accA, degp, bstk, wf):
    return pl.pallas_call(
        _mid_body,
        grid=(NB,),
        in_specs=[
            pl.BlockSpec((2, BN, C), lambda j: (0, j, 0)),
            pl.BlockSpec((32, BN), lambda j: (0, j)),
            pl.BlockSpec((2, 1, C), lambda j: (0, 0, 0)),
            pl.BlockSpec((D, C), lambda j: (0, 0)),
        ],
        out_specs=pl.BlockSpec((2, BN, C // 2), lambda j: (0, j, 0)),
        out_shape=jax.ShapeDtypeStruct((2, NP, C // 2), jnp.float32),
    )(accA, degp, bstk, wf)


def _final_body(acc_ref, degp_ref, bf_ref, out_ref):
    dinv, _ = _dinv_of(degp_ref[...])
    y = jnp.concatenate([acc_ref[0], acc_ref[1]], axis=-1)
    out_ref[...] = y * dinv[:, None] + bf_ref[0]


def _tc_final(accB, degp, bf2d):
    return pl.pallas_call(
        _final_body,
        grid=(NB,),
        in_specs=[
            pl.BlockSpec((2, BN, C // 2), lambda j: (0, j, 0)),
            pl.BlockSpec((32, BN), lambda j: (0, j)),
            pl.BlockSpec((1, C), lambda j: (0, 0)),
        ],
        out_specs=pl.BlockSpec((BN, C), lambda j: (j, 0)),
        out_shape=jax.ShapeDtypeStruct((NP, C), jnp.float32),
    )(accB, degp, bf2d)


def kernel(x_list, edge_index, W0, b0, W1, b1, Wf, bf):
    src = edge_index[0]
    dst = edge_index[1]
    pad = jnp.full((EP - E,), NP - 1, dtype=jnp.int32)
    srcp = jnp.concatenate([src, pad])
    dstp = jnp.concatenate([dst, pad])
    srcA = srcp.reshape(NTILES, NCHT, CH)
    dstA = dstp.reshape(NTILES, NCHT, CH)
    dst32 = dstp.reshape(32, EPW32)

    xp = jnp.pad(x_list, ((0, 0), (0, NP - N), (0, 0)))
    wstk = jnp.stack([W0, W1])
    bstk = jnp.stack([b0, b1])[:, None, :]

    degp = _deg_kernel(dst32)
    u, dinv2 = _tc_prep(xp, wstk, degp)
    accA, _ = _phase_a(u.reshape(2 * NP, C // 2), srcA, dstA, dinv2)
    uz = _tc_mid(accA.reshape(2, NP, C), degp, bstk, Wf)
    accB = _phase_b(uz.reshape(2 * NP, C // 2), srcA, dstA, dinv2)
    out = _tc_final(accB.reshape(2, NP, C // 2), degp, bf[None, :])
    return out[:N]
